# baseline (speedup 1.0000x reference)
import functools

import jax
import jax.numpy as jnp
from jax import lax
from jax.experimental import pallas as pl
from jax.experimental.pallas import tpu as pltpu
from jax.experimental.pallas import tpu_sc as plsc

N_NODES = 10000
N_EDGES = 320000
D = 128
DA = 144

NC = 2
NS = 16
HALF = N_NODES // NC
NPC = 5120
E_PER_T = N_EDGES // NS
SEG = 10000
N_SEG = E_PER_T // SEG
CHUNK = 80
N_CHUNKS = SEG // CHUNK
STRIPE = NPC // NS
ZR = 80
DEG_W = 16
V = 16


def _sc_aggregate(feat_aug, src_e, dst_e, zero_acc):
    mesh = plsc.VectorSubcoreMesh(core_axis_name="c", subcore_axis_name="s")

    @functools.partial(
        pl.kernel,
        out_type=jax.ShapeDtypeStruct((NC, NPC, DA), jnp.float32),
        mesh=mesh,
        compiler_params=pltpu.CompilerParams(use_tc_tiling_on_sc=False),
        scratch_types=[
            pltpu.VMEM((SEG,), jnp.int32),
            pltpu.VMEM((SEG,), jnp.int32),
            pltpu.VMEM((CHUNK,), jnp.int32),
            pltpu.VMEM((CHUNK, DA), jnp.float32),
            pltpu.VMEM((CHUNK, DA), jnp.float32),
            pltpu.VMEM_SHARED((NPC, DA), jnp.float32),
            pltpu.SemaphoreType.DMA,
            pltpu.SemaphoreType.DMA,
        ],
    )
    def body(feat_hbm, src_hbm, dst_hbm, zacc_hbm, acc_out,
             src_v, dst_v, chunk_v, rows_a, rows_b, acc_sh, sem_a, sem_b):
        cid = lax.axis_index("c")
        sid = lax.axis_index("s")
        lo = cid * HALF

        for q in range(STRIPE // ZR):
            sub = pl.ds(sid * STRIPE + q * ZR, ZR)
            pltpu.sync_copy(zacc_hbm, acc_sh.at[sub])
        plsc.subcore_barrier()

        lane = lax.iota(jnp.int32, V)

        def remap(j):
            for v in range(CHUNK // V):
                local = dst_v[pl.ds(j * CHUNK + v * V, V)] - lo
                ok = (local >= 0) & (local < HALF)
                dump = HALF + ((j + v) & 63) + lane
                chunk_v[pl.ds(v * V, V)] = jnp.where(ok, local, dump)

        def gsrc(j):
            return feat_hbm.at[src_v.at[pl.ds(j * CHUNK, CHUNK)]]

        def pair_step(p, carry):
            j0 = 2 * p
            j1 = j0 + 1
            pltpu.async_copy(gsrc(j1), rows_b, sem_b)
            pltpu.make_async_copy(gsrc(j0), rows_a, sem_a).wait()
            remap(j0)
            pltpu.sync_copy(rows_a, acc_sh.at[chunk_v], add=True)
            pltpu.async_copy(gsrc(j0 + 2), rows_a, sem_a)
            pltpu.make_async_copy(gsrc(j1), rows_b, sem_b).wait()
            remap(j1)
            pltpu.sync_copy(rows_b, acc_sh.at[chunk_v], add=True)
            return carry

        last = N_CHUNKS - 1
        for s in range(N_SEG):
            base = sid * E_PER_T + s * SEG
            pltpu.sync_copy(src_hbm.at[pl.ds(base, SEG)], src_v)
            pltpu.sync_copy(dst_hbm.at[pl.ds(base, SEG)], dst_v)
            pltpu.async_copy(gsrc(0), rows_a, sem_a)
            lax.fori_loop(0, N_CHUNKS // 2, pair_step, 0)
            pltpu.make_async_copy(gsrc(last), rows_a, sem_a).wait()
            remap(last)
            pltpu.sync_copy(rows_a, acc_sh.at[chunk_v], add=True)

        plsc.subcore_barrier()
        for q in range(STRIPE // ZR):
            sub = pl.ds(sid * STRIPE + q * ZR, ZR)
            pltpu.sync_copy(acc_sh.at[sub], acc_out.at[cid, sub])

    return body(feat_aug, src_e, dst_e, zero_acc)


def _tc_finish_body(acc_ref, wt_ref, b_ref, out_ref):
    a = acc_ref[0]
    h = a[:, :D] / jnp.maximum(a[:, D:D + 1], 1.0)
    o = jnp.dot(h, wt_ref[...], preferred_element_type=jnp.float32)
    out_ref[...] = jnp.maximum(o + b_ref[...], 0.0)


def _tc_finish(acc, wt, b2d):
    R = 200
    nb = HALF // R
    return pl.pallas_call(
        _tc_finish_body,
        grid=(NC, nb),
        in_specs=[
            pl.BlockSpec((1, R, DA), lambda c, i: (c, i, 0)),
            pl.BlockSpec((D, D), lambda c, i: (0, 0)),
            pl.BlockSpec((1, D), lambda c, i: (0, 0)),
        ],
        out_specs=pl.BlockSpec((R, D), lambda c, i: (c * nb + i, 0)),
        out_shape=jax.ShapeDtypeStruct((N_NODES, D), jnp.float32),
    )(acc, wt, b2d)


def kernel(feature, edge_index, W, b):
    src_e = edge_index[0].astype(jnp.int32)
    dst_e = edge_index[1].astype(jnp.int32)
    feat_aug = jnp.concatenate(
        [feature,
         jnp.ones((N_NODES, 1), jnp.float32),
         jnp.zeros((N_NODES, DA - D - 1), jnp.float32)], axis=1)
    zero_acc = jnp.zeros((ZR, DA), jnp.float32)

    acc = _sc_aggregate(feat_aug, src_e, dst_e, zero_acc)
    return _tc_finish(acc, W.T, b.reshape(1, D))
